# baseline (speedup 1.0000x reference)
import functools

import jax
import jax.numpy as jnp
from jax import lax
from jax.experimental import pallas as pl
from jax.experimental.pallas import tpu as pltpu
from jax.experimental.pallas import tpu_sc as plsc

_NC = 2
_NS = 16
_NW = _NC * _NS

_BB = 128


@functools.partial(jax.jit, static_argnames=("b", "l", "d"))
def _sc_gather(idx_t, table, b, l, d):
    b_per_w = b // _NW
    nbb = b_per_w // _BB
    n_chunks = nbb * l
    dh = d // 8
    mesh = plsc.VectorSubcoreMesh(core_axis_name="c", subcore_axis_name="s")

    @functools.partial(
        pl.kernel,
        out_type=jax.ShapeDtypeStruct((l, dh, b // _BB, 8, _BB), jnp.float32),
        mesh=mesh,
        compiler_params=pltpu.CompilerParams(
            use_tc_tiling_on_sc=False, needs_layout_passes=False
        ),
        scratch_types=[
            pltpu.VMEM((l, b_per_w), jnp.int32),
            pltpu.VMEM((2, _BB, d), jnp.float32),
            pltpu.VMEM((2, dh, 8, _BB), jnp.float32),
        ]
        + [pltpu.SemaphoreType.DMA] * 4,
    )
    def k(idx_hbm, table_hbm, out_hbm, idx_v, gbuf, tbuf, *sems):
        gsems = sems[:2]
        wsems = sems[2:]
        wid = lax.axis_index("s") * _NC + lax.axis_index("c")
        b0 = wid * b_per_w
        pltpu.sync_copy(idx_hbm.at[:, pl.ds(b0, b_per_w)], idx_v)

        iota = lax.iota(jnp.int32, 16)

        def idx_at(c):
            bbl = c // l
            li = c - bbl * l
            return idx_v.at[li].at[pl.ds(bbl * _BB, _BB)]

        def out_at(c):
            bbl = c // l
            li = c - bbl * l
            return out_hbm.at[li].at[:, wid * nbb + bbl]

        def transpose(db):
            src = gbuf.at[db]
            dst = tbuf.at[db]
            for hi in range(8):
                for lo in range(8):
                    col = jnp.full((16,), hi * 8 + lo, jnp.int32)
                    for bv in range(8):
                        v = plsc.load_gather(src, [iota + bv * 16, col])
                        dst[hi, lo, pl.ds(bv * 16, 16)] = v

        pltpu.async_copy(table_hbm.at[idx_at(0)], gbuf.at[0], gsems[0])
        pltpu.async_copy(table_hbm.at[idx_at(1)], gbuf.at[1], gsems[1])

        def outer(j, carry):
            c0 = j * 2
            for db in range(2):
                c = c0 + db
                pltpu.make_async_copy(
                    table_hbm.at[idx_at(c)], gbuf.at[db], gsems[db]
                ).wait()

                @pl.when(c >= 2)
                def _drain_wb():
                    pltpu.make_async_copy(
                        tbuf.at[db], out_at(0), wsems[db]
                    ).wait()

                transpose(db)

                @pl.when(c + 2 < n_chunks)
                def _launch():
                    pltpu.async_copy(
                        table_hbm.at[idx_at(c + 2)], gbuf.at[db], gsems[db]
                    )

                pltpu.async_copy(tbuf.at[db], out_at(c), wsems[db])

            return carry

        lax.fori_loop(0, n_chunks // 2, outer, 0)

        for db in range(2):
            pltpu.make_async_copy(tbuf.at[db], out_at(0), wsems[db]).wait()

    return k(idx_t, table)


def kernel(x, weights):
    b, l = x.shape
    d = weights.shape[1]
    idx_t = x.T.astype(jnp.int32)
    out5 = _sc_gather(idx_t, weights, b, l, d)
    return out5.transpose(2, 4, 0, 1, 3).reshape(b, l, d)


# speedup vs baseline: 1.9829x; 1.9829x over previous
import functools

import jax
import jax.numpy as jnp
from jax import lax
from jax.experimental import pallas as pl
from jax.experimental.pallas import tpu as pltpu
from jax.experimental.pallas import tpu_sc as plsc

_NC = 2
_NS = 16
_NW = _NC * _NS

_BB = 128
_PW = 129


@functools.partial(jax.jit, static_argnames=("b", "l", "d"))
def _sc_gather(idx_t, table, b, l, d):
    b_per_w = b // _NW
    nbb = b_per_w // _BB
    n_chunks = nbb * l
    dh = d // 8
    mesh = plsc.VectorSubcoreMesh(core_axis_name="c", subcore_axis_name="s")

    @functools.partial(
        pl.kernel,
        out_type=jax.ShapeDtypeStruct((l, dh, b // _BB, 8, _BB), jnp.float32),
        mesh=mesh,
        compiler_params=pltpu.CompilerParams(
            use_tc_tiling_on_sc=False, needs_layout_passes=False
        ),
        scratch_types=[
            pltpu.VMEM((l, b_per_w), jnp.int32),
            pltpu.VMEM((2, _BB, d), jnp.float32),
            pltpu.VMEM((2, dh, 8, _PW), jnp.float32),
        ]
        + [pltpu.SemaphoreType.DMA] * 4,
    )
    def k(idx_hbm, table_hbm, out_hbm, idx_v, gbuf, tbuf, *sems):
        gsems = sems[:2]
        wsems = sems[2:]
        wid = lax.axis_index("s") * _NC + lax.axis_index("c")
        b0 = wid * b_per_w
        pltpu.sync_copy(idx_hbm.at[:, pl.ds(b0, b_per_w)], idx_v)

        iota = lax.iota(jnp.int32, 16)

        def idx_at(c):
            bbl = c // l
            li = c - bbl * l
            return idx_v.at[li].at[pl.ds(bbl * _BB, _BB)]

        def out_at(c):
            bbl = c // l
            li = c - bbl * l
            return out_hbm.at[li].at[:, wid * nbb + bbl]

        d_hi = [(lax.iota(jnp.int32, 16) + dv * 16) >> 3 for dv in range(4)]
        d_lo = [(lax.iota(jnp.int32, 16) + dv * 16) & 7 for dv in range(4)]

        def transpose(db):
            src = gbuf.at[db]
            dst = tbuf.at[db]

            def row(bi, carry):
                bvec = jnp.full((16,), bi, jnp.int32)
                for dv in range(4):
                    v = src[bi, pl.ds(dv * 16, 16)]
                    plsc.store_scatter(dst, [d_hi[dv], d_lo[dv], bvec], v)
                return carry

            lax.fori_loop(0, _BB, row, 0)

        pltpu.async_copy(table_hbm.at[idx_at(0)], gbuf.at[0], gsems[0])
        pltpu.async_copy(table_hbm.at[idx_at(1)], gbuf.at[1], gsems[1])

        def outer(j, carry):
            c0 = j * 2
            for db in range(2):
                c = c0 + db
                pltpu.make_async_copy(
                    table_hbm.at[idx_at(c)], gbuf.at[db], gsems[db]
                ).wait()

                @pl.when(c >= 2)
                def _drain_wb():
                    pltpu.make_async_copy(
                        tbuf.at[db].at[:, :, pl.ds(0, _BB)], out_at(0), wsems[db]
                    ).wait()

                transpose(db)

                @pl.when(c + 2 < n_chunks)
                def _launch():
                    pltpu.async_copy(
                        table_hbm.at[idx_at(c + 2)], gbuf.at[db], gsems[db]
                    )

                pltpu.async_copy(
                    tbuf.at[db].at[:, :, pl.ds(0, _BB)], out_at(c), wsems[db]
                )

            return carry

        lax.fori_loop(0, n_chunks // 2, outer, 0)

        for db in range(2):
            pltpu.make_async_copy(
                tbuf.at[db].at[:, :, pl.ds(0, _BB)], out_at(0), wsems[db]
            ).wait()

    return k(idx_t, table)


def kernel(x, weights):
    b, l = x.shape
    d = weights.shape[1]
    idx_t = x.T.astype(jnp.int32)
    out5 = _sc_gather(idx_t, weights, b, l, d)
    return out5.transpose(2, 4, 0, 1, 3).reshape(b, l, d)


# speedup vs baseline: 2.2736x; 1.1466x over previous
import functools

import jax
import jax.numpy as jnp
from jax import lax
from jax.experimental import pallas as pl
from jax.experimental.pallas import tpu as pltpu
from jax.experimental.pallas import tpu_sc as plsc

_NC = 2
_NS = 16
_NW = _NC * _NS

_BB = 128
_VB = 2048
_PW = 129


def _tc_detile(w_t, v, d):
    def body(in_ref, out_ref):
        out_ref[:, 0:d] = in_ref[...].T

    return pl.pallas_call(
        body,
        grid=(pl.cdiv(v, _VB),),
        in_specs=[pl.BlockSpec((d, _VB), lambda i: (0, i))],
        out_specs=pl.BlockSpec((_VB, 2 * d), lambda i: (i, 0)),
        out_shape=jax.ShapeDtypeStruct((v, 2 * d), jnp.float32),
    )(w_t)


@functools.partial(jax.jit, static_argnames=("b", "l", "d"))
def _sc_gather(idx_t, table, b, l, d):
    b_per_w = b // _NW
    nbb = b_per_w // _BB
    n_chunks = nbb * l
    dh = d // 8
    mesh = plsc.VectorSubcoreMesh(core_axis_name="c", subcore_axis_name="s")

    @functools.partial(
        pl.kernel,
        out_type=jax.ShapeDtypeStruct((l, dh, b // _BB, 8, _BB), jnp.float32),
        mesh=mesh,
        compiler_params=pltpu.CompilerParams(
            use_tc_tiling_on_sc=False, needs_layout_passes=False
        ),
        scratch_types=[
            pltpu.VMEM((l, b_per_w), jnp.int32),
            pltpu.VMEM((2, _BB, 2 * d), jnp.float32),
            pltpu.VMEM((2, dh, 8, _PW), jnp.float32),
        ]
        + [pltpu.SemaphoreType.DMA] * 4,
    )
    def k(idx_hbm, table_hbm, out_hbm, idx_v, gbuf, tbuf, *sems):
        gsems = sems[:2]
        wsems = sems[2:]
        wid = lax.axis_index("s") * _NC + lax.axis_index("c")
        b0 = wid * b_per_w
        pltpu.sync_copy(idx_hbm.at[:, pl.ds(b0, b_per_w)], idx_v)

        iota = lax.iota(jnp.int32, 16)

        def idx_at(c):
            bbl = c // l
            li = c - bbl * l
            return idx_v.at[li].at[pl.ds(bbl * _BB, _BB)]

        def out_at(c):
            bbl = c // l
            li = c - bbl * l
            return out_hbm.at[li].at[:, wid * nbb + bbl]

        d_hi = [(lax.iota(jnp.int32, 16) + dv * 16) >> 3 for dv in range(4)]
        d_lo = [(lax.iota(jnp.int32, 16) + dv * 16) & 7 for dv in range(4)]

        def transpose(db):
            src = gbuf.at[db]
            dst = tbuf.at[db]

            def row(bi, carry):
                bvec = jnp.full((16,), bi, jnp.int32)
                for dv in range(4):
                    v = src[bi, pl.ds(dv * 16, 16)]
                    plsc.store_scatter(dst, [d_hi[dv], d_lo[dv], bvec], v)
                return carry

            lax.fori_loop(0, _BB, row, 0)

        pltpu.async_copy(table_hbm.at[idx_at(0)], gbuf.at[0], gsems[0])
        pltpu.async_copy(table_hbm.at[idx_at(1)], gbuf.at[1], gsems[1])

        def outer(j, carry):
            c0 = j * 2
            for db in range(2):
                c = c0 + db
                pltpu.make_async_copy(
                    table_hbm.at[idx_at(c)], gbuf.at[db], gsems[db]
                ).wait()

                @pl.when(c >= 2)
                def _drain_wb():
                    pltpu.make_async_copy(
                        tbuf.at[db].at[:, :, pl.ds(0, _BB)], out_at(0), wsems[db]
                    ).wait()

                transpose(db)

                @pl.when(c + 2 < n_chunks)
                def _launch():
                    pltpu.async_copy(
                        table_hbm.at[idx_at(c + 2)], gbuf.at[db], gsems[db]
                    )

                pltpu.async_copy(
                    tbuf.at[db].at[:, :, pl.ds(0, _BB)], out_at(c), wsems[db]
                )

            return carry

        lax.fori_loop(0, n_chunks // 2, outer, 0)

        for db in range(2):
            pltpu.make_async_copy(
                tbuf.at[db].at[:, :, pl.ds(0, _BB)], out_at(0), wsems[db]
            ).wait()

    return k(idx_t, table)


def kernel(x, weights):
    b, l = x.shape
    v, d = weights.shape
    idx_t = x.T.astype(jnp.int32)
    table = _tc_detile(weights.T, v, d)
    out5 = _sc_gather(idx_t, table, b, l, d)
    return out5.transpose(2, 4, 0, 1, 3).reshape(b, l, d)
